# baseline (speedup 1.0000x reference)
import jax
import jax.numpy as jnp
import numpy as np
from jax.experimental import pallas as pl
from jax.experimental.pallas import tpu as pltpu


def _mask_kernel(a_ref, g_ref, w_ref):
    a = a_ref[...]
    g = g_ref[0]
    t = jnp.dot(a, g, preferred_element_type=jnp.float32)
    r = jax.lax.dot_general(t, a, (((1,), (1,)), ((), ())),
                            preferred_element_type=jnp.float32)
    w_ref[0] = (r >= 1.0).astype(jnp.float32)


def _stats_kernel(x_ref, w_ref, o_ref):
    x = x_ref[0, 0]
    wv = w_ref[0]
    xm = x * wv
    pos = wv > 0.0
    neg_inf = jnp.float32(-jnp.inf)
    pos_inf = jnp.float32(jnp.inf)
    s1 = jnp.sum(xm, axis=0)
    s2 = jnp.sum(xm * xm, axis=0)
    sa = jnp.sum(jnp.abs(xm), axis=0)
    mx = jnp.max(jnp.where(pos, x, neg_inf), axis=0)
    mn = jnp.min(jnp.where(pos, x, pos_inf), axis=0)
    cnt = jnp.sum(wv)
    cs = jnp.maximum(cnt, 1.0)
    mean = s1 / cs
    var = (s2 - 2.0 * mean * s1 + cnt * mean * mean) / jnp.maximum(cnt - 1.0, 1.0)
    o_ref[0, 0, 0, :] = mean
    o_ref[0, 0, 1, :] = mean
    o_ref[0, 0, 2, :] = var
    o_ref[0, 0, 3, :] = mx
    o_ref[0, 0, 4, :] = mn
    o_ref[0, 0, 5, :] = sa


def kernel(feature_maps, gts):
    L, N, C, H, W = feature_maps.shape
    P = H * W
    S = gts.shape[-1]
    a = jax.image.resize(jnp.eye(S, dtype=jnp.float32), (H, S), method="bilinear")

    g = gts.reshape(N, S, S)
    w = pl.pallas_call(
        _mask_kernel,
        grid=(N,),
        in_specs=[
            pl.BlockSpec((H, S), lambda n: (0, 0)),
            pl.BlockSpec((1, S, S), lambda n: (n, 0, 0)),
        ],
        out_specs=pl.BlockSpec((1, H, W), lambda n: (n, 0, 0)),
        out_shape=jax.ShapeDtypeStruct((N, H, W), jnp.float32),
    )(a, g)

    w2 = w.reshape(N, P, 1)
    x = feature_maps.transpose(0, 1, 3, 4, 2).reshape(L, N, P, C)
    o = pl.pallas_call(
        _stats_kernel,
        grid=(N, L),
        in_specs=[
            pl.BlockSpec((1, 1, P, C), lambda n, l: (l, n, 0, 0)),
            pl.BlockSpec((1, P, 1), lambda n, l: (n, 0, 0)),
        ],
        out_specs=pl.BlockSpec((1, 1, 6, C), lambda n, l: (l, n, 0, 0)),
        out_shape=jax.ShapeDtypeStruct((L, N, 6, C), jnp.float32),
    )(x, w2)

    return o.transpose(1, 0, 2, 3).reshape(N, L * 6 * C)


# speedup vs baseline: 1.2175x; 1.2175x over previous
import jax
import jax.numpy as jnp
import numpy as np
from jax.experimental import pallas as pl
from jax.experimental.pallas import tpu as pltpu


def _mask_kernel(a_ref, g_ref, w_ref):
    a = a_ref[...]
    g = g_ref[0]
    t = jnp.dot(a, g, preferred_element_type=jnp.float32)
    r = jax.lax.dot_general(t, a, (((1,), (1,)), ((), ())),
                            preferred_element_type=jnp.float32)
    w_ref[0] = (r >= 1.0).astype(jnp.float32)


def _stats_kernel(x_ref, w_ref, o_ref):
    c_dim = o_ref.shape[3]
    wv = w_ref[0]
    cnt = jnp.sum(wv)
    neg_inf = jnp.float32(-jnp.inf)
    pos_inf = jnp.float32(jnp.inf)

    @pl.when(cnt > 0.0)
    def _dense():
        x = x_ref[0, 0]
        xm = x * wv
        pos = wv > 0.0
        s1 = jnp.sum(xm, axis=0)
        s2 = jnp.sum(xm * xm, axis=0)
        sa = jnp.sum(jnp.abs(xm), axis=0)
        mx = jnp.max(jnp.where(pos, x, neg_inf), axis=0)
        mn = jnp.min(jnp.where(pos, x, pos_inf), axis=0)
        cs = jnp.maximum(cnt, 1.0)
        mean = s1 / cs
        var = (s2 - 2.0 * mean * s1 + cnt * mean * mean) / jnp.maximum(cnt - 1.0, 1.0)
        o_ref[0, 0, 0, :] = mean
        o_ref[0, 0, 1, :] = mean
        o_ref[0, 0, 2, :] = var
        o_ref[0, 0, 3, :] = mx
        o_ref[0, 0, 4, :] = mn
        o_ref[0, 0, 5, :] = sa

    @pl.when(cnt <= 0.0)
    def _empty():
        zeros = jnp.zeros((3, c_dim), jnp.float32)
        o_ref[0, 0, 0:3, :] = zeros
        o_ref[0, 0, 3, :] = jnp.full((c_dim,), neg_inf)
        o_ref[0, 0, 4, :] = jnp.full((c_dim,), pos_inf)
        o_ref[0, 0, 5, :] = jnp.zeros((c_dim,), jnp.float32)


def kernel(feature_maps, gts):
    L, N, C, H, W = feature_maps.shape
    P = H * W
    S = gts.shape[-1]
    a = jax.image.resize(jnp.eye(S, dtype=jnp.float32), (H, S), method="bilinear")

    g = gts.reshape(N, S, S)
    w = pl.pallas_call(
        _mask_kernel,
        grid=(N,),
        in_specs=[
            pl.BlockSpec((H, S), lambda n: (0, 0)),
            pl.BlockSpec((1, S, S), lambda n: (n, 0, 0)),
        ],
        out_specs=pl.BlockSpec((1, H, W), lambda n: (n, 0, 0)),
        out_shape=jax.ShapeDtypeStruct((N, H, W), jnp.float32),
    )(a, g)

    w2 = w.reshape(N, P, 1)
    x = feature_maps.transpose(0, 1, 3, 4, 2).reshape(L, N, P, C)
    o = pl.pallas_call(
        _stats_kernel,
        grid=(N, L),
        in_specs=[
            pl.BlockSpec((1, 1, P, C), lambda n, l: (l, n, 0, 0)),
            pl.BlockSpec((1, P, 1), lambda n, l: (n, 0, 0)),
        ],
        out_specs=pl.BlockSpec((1, 1, 6, C), lambda n, l: (l, n, 0, 0)),
        out_shape=jax.ShapeDtypeStruct((L, N, 6, C), jnp.float32),
    )(x, w2)

    return o.transpose(1, 0, 2, 3).reshape(N, L * 6 * C)


# speedup vs baseline: 1.4838x; 1.2187x over previous
import jax
import jax.numpy as jnp
import numpy as np
from jax.experimental import pallas as pl
from jax.experimental.pallas import tpu as pltpu


def _mask_kernel(a_ref, g_ref, w_ref):
    a = a_ref[...]
    g = g_ref[0]
    t = jnp.dot(a, g, preferred_element_type=jnp.float32)
    r = jax.lax.dot_general(t, a, (((1,), (1,)), ((), ())),
                            preferred_element_type=jnp.float32)
    w_ref[0] = (r >= 1.0).astype(jnp.float32)


def _stats_kernel(nidx_ref, x_ref, w_ref, o_ref):
    c_dim = o_ref.shape[3]
    wv = w_ref[0]
    cnt = jnp.sum(wv)
    neg_inf = jnp.float32(-jnp.inf)
    pos_inf = jnp.float32(jnp.inf)

    @pl.when(cnt > 0.0)
    def _dense():
        x = x_ref[0, 0]
        xm = x * wv
        pos = wv > 0.0
        s1 = jnp.sum(xm, axis=0)
        s2 = jnp.sum(xm * xm, axis=0)
        sa = jnp.sum(jnp.abs(xm), axis=0)
        mx = jnp.max(jnp.where(pos, x, neg_inf), axis=0)
        mn = jnp.min(jnp.where(pos, x, pos_inf), axis=0)
        cs = jnp.maximum(cnt, 1.0)
        mean = s1 / cs
        var = (s2 - 2.0 * mean * s1 + cnt * mean * mean) / jnp.maximum(cnt - 1.0, 1.0)
        o_ref[0, 0, 0, :] = mean
        o_ref[0, 0, 1, :] = mean
        o_ref[0, 0, 2, :] = var
        o_ref[0, 0, 3, :] = mx
        o_ref[0, 0, 4, :] = mn
        o_ref[0, 0, 5, :] = sa

    @pl.when(cnt <= 0.0)
    def _empty():
        zeros = jnp.zeros((3, c_dim), jnp.float32)
        o_ref[0, 0, 0:3, :] = zeros
        o_ref[0, 0, 3, :] = jnp.full((c_dim,), neg_inf)
        o_ref[0, 0, 4, :] = jnp.full((c_dim,), pos_inf)
        o_ref[0, 0, 5, :] = jnp.zeros((c_dim,), jnp.float32)


def kernel(feature_maps, gts):
    L, N, C, H, W = feature_maps.shape
    P = H * W
    S = gts.shape[-1]
    a = jax.image.resize(jnp.eye(S, dtype=jnp.float32), (H, S), method="bilinear")

    g = gts.reshape(N, S, S)
    w = pl.pallas_call(
        _mask_kernel,
        grid=(N,),
        in_specs=[
            pl.BlockSpec((H, S), lambda n: (0, 0)),
            pl.BlockSpec((1, S, S), lambda n: (n, 0, 0)),
        ],
        out_specs=pl.BlockSpec((1, H, W), lambda n: (n, 0, 0)),
        out_shape=jax.ShapeDtypeStruct((N, H, W), jnp.float32),
    )(a, g)

    w2 = w.reshape(N, P, 1)
    nonempty = w2.sum(axis=(1, 2)) > 0.0
    nidx = jnp.where(nonempty, jnp.arange(N, dtype=jnp.int32), 0)
    x = feature_maps.transpose(0, 1, 3, 4, 2).reshape(L, N, P, C)
    o = pl.pallas_call(
        _stats_kernel,
        grid_spec=pltpu.PrefetchScalarGridSpec(
            num_scalar_prefetch=1,
            grid=(L, N),
            in_specs=[
                pl.BlockSpec((1, 1, P, C), lambda l, n, nidx: (l, nidx[n], 0, 0)),
                pl.BlockSpec((1, P, 1), lambda l, n, nidx: (n, 0, 0)),
            ],
            out_specs=pl.BlockSpec((1, 1, 6, C), lambda l, n, nidx: (l, n, 0, 0)),
        ),
        out_shape=jax.ShapeDtypeStruct((L, N, 6, C), jnp.float32),
    )(nidx, x, w2)

    return o.transpose(1, 0, 2, 3).reshape(N, L * 6 * C)


# speedup vs baseline: 1.4879x; 1.0028x over previous
import jax
import jax.numpy as jnp
from jax.experimental import pallas as pl
from jax.experimental.pallas import tpu as pltpu


def _mask_kernel(a_ref, g_ref, w_ref):
    a = a_ref[...]
    g = g_ref[0]
    t = jnp.dot(a, g, preferred_element_type=jnp.float32)
    r = jax.lax.dot_general(t, a, (((1,), (1,)), ((), ())),
                            preferred_element_type=jnp.float32)
    w_ref[0] = (r >= 1.0).astype(jnp.float32)


def _stats_kernel(nidx_ref, x_ref, w_ref, o_ref):
    c_dim = o_ref.shape[3]
    wv = w_ref[0]
    cnt = jnp.sum(wv)
    neg_inf = jnp.float32(-jnp.inf)
    pos_inf = jnp.float32(jnp.inf)

    @pl.when(cnt > 0.0)
    def _dense():
        x = x_ref[0, 0]
        xm = x * wv
        pos = wv > 0.0
        s1 = jnp.sum(xm, axis=0)
        s2 = jnp.sum(xm * xm, axis=0)
        sa = jnp.sum(jnp.abs(xm), axis=0)
        mx = jnp.max(jnp.where(pos, x, neg_inf), axis=0)
        mn = jnp.min(jnp.where(pos, x, pos_inf), axis=0)
        cs = jnp.maximum(cnt, 1.0)
        mean = s1 / cs
        var = (s2 - 2.0 * mean * s1 + cnt * mean * mean) / jnp.maximum(cnt - 1.0, 1.0)
        o_ref[0, 0, 0, :] = mean
        o_ref[0, 0, 1, :] = mean
        o_ref[0, 0, 2, :] = var
        o_ref[0, 0, 3, :] = mx
        o_ref[0, 0, 4, :] = mn
        o_ref[0, 0, 5, :] = sa

    @pl.when(cnt <= 0.0)
    def _empty():
        zeros = jnp.zeros((3, c_dim), jnp.float32)
        o_ref[0, 0, 0:3, :] = zeros
        o_ref[0, 0, 3, :] = jnp.full((c_dim,), neg_inf)
        o_ref[0, 0, 4, :] = jnp.full((c_dim,), pos_inf)
        o_ref[0, 0, 5, :] = jnp.zeros((c_dim,), jnp.float32)


def kernel(feature_maps, gts):
    L, N, C, H, W = feature_maps.shape
    P = H * W
    S = gts.shape[-1]
    a = jax.image.resize(jnp.eye(S, dtype=jnp.float32), (H, S), method="bilinear")

    g = gts.reshape(N, S, S)
    w = pl.pallas_call(
        _mask_kernel,
        grid=(N,),
        in_specs=[
            pl.BlockSpec((H, S), lambda n: (0, 0)),
            pl.BlockSpec((1, S, S), lambda n: (n, 0, 0)),
        ],
        out_specs=pl.BlockSpec((1, H, W), lambda n: (n, 0, 0)),
        out_shape=jax.ShapeDtypeStruct((N, H, W), jnp.float32),
    )(a, g)

    w2 = w.reshape(N, P, 1)
    nonempty = w2.sum(axis=(1, 2)) > 0.0
    nidx = jnp.where(nonempty, jnp.arange(N, dtype=jnp.int32), 0)
    x = feature_maps.transpose(0, 1, 3, 4, 2).reshape(L, N, P, C)
    o = pl.pallas_call(
        _stats_kernel,
        grid_spec=pltpu.PrefetchScalarGridSpec(
            num_scalar_prefetch=1,
            grid=(L, N),
            in_specs=[
                pl.BlockSpec((1, 1, P, C), lambda l, n, nidx: (l, nidx[n], 0, 0)),
                pl.BlockSpec((1, P, 1), lambda l, n, nidx: (n, 0, 0)),
            ],
            out_specs=pl.BlockSpec((1, 1, 6, C), lambda l, n, nidx: (n, l, 0, 0)),
        ),
        out_shape=jax.ShapeDtypeStruct((N, L, 6, C), jnp.float32),
    )(nidx, x, w2)

    return o.reshape(N, L * 6 * C)
